# baseline (speedup 1.0000x reference)
import functools
import numpy as np
import jax
import jax.numpy as jnp
from jax import lax
from jax.experimental import pallas as pl
from jax.experimental.pallas import tpu as pltpu
from jax.experimental.pallas import tpu_sc as plsc

BLANK = 9999
PAD = 0
LS = 0.1
CONF = 1.0 - LS
W = 0.5
NEG = -1e30

NC = 2
NS = 16
NW = NC * NS


def _att_body(scores_ref, labels_ref, out_ref, *, K, smooth):
    x = scores_ref[...]
    g = labels_ref[0]
    R, V = x.shape
    rowsum = jnp.sum(x, axis=1, keepdims=True)
    s0 = x[:, 0:1]
    iota_v = lax.broadcasted_iota(jnp.int32, (R, V), 1)
    sg = jnp.sum(jnp.where(iota_v == g, x, 0.0), axis=1, keepdims=True)
    att = jnp.where(g == PAD, 0.0,
                    K - CONF * sg - smooth * (rowsum - s0 - sg))
    part = jnp.sum(att)

    @pl.when(pl.program_id(0) == 0)
    def _():
        out_ref[0, 0] = 0.0

    out_ref[0, 0] += part


def _att_loss(scores, labels_flat):
    N, V = scores.shape
    R = 128
    nblk = N // R
    smooth = LS / (V - 2)
    K = float(CONF * np.log(CONF) + (V - 2) * smooth * np.log(smooth))
    labels3 = labels_flat.reshape(nblk, R, 1).astype(jnp.int32)
    out = pl.pallas_call(
        functools.partial(_att_body, K=K, smooth=smooth),
        grid=(nblk,),
        in_specs=[
            pl.BlockSpec((R, V), lambda i: (i, 0)),
            pl.BlockSpec((1, R, 1), lambda i: (i, 0, 0)),
        ],
        out_specs=pl.BlockSpec(memory_space=pltpu.SMEM),
        out_shape=jax.ShapeDtypeStruct((1, 1), jnp.float32),
    )(scores, labels3)
    return out[0, 0]


def _sc_gather(enc_flat, tgt, T, B, V, S):
    PPW = (T * B) // NW
    ROW = 80
    SC16 = S // 16

    mesh = plsc.VectorSubcoreMesh(core_axis_name="c", subcore_axis_name="s")

    @functools.partial(
        pl.kernel,
        out_type=jax.ShapeDtypeStruct((T * B * ROW,), jnp.float32),
        mesh=mesh,
        scratch_types=[
            pltpu.VMEM((B, S), jnp.int32),
            pltpu.VMEM((PPW * ROW,), jnp.int32),
            pltpu.VMEM((PPW * ROW,), jnp.float32),
            pltpu.SemaphoreType.DMA,
        ],
    )
    def k(enc_hbm, tgt_hbm, out_hbm, tgt_v, idx_t, val_t, sem):
        wid = lax.axis_index("s") * NC + lax.axis_index("c")
        p0 = wid * PPW

        pltpu.sync_copy(tgt_hbm, tgt_v)

        TT = T // 128
        BOFF = (BLANK >> 3) * (1024 * TT) + (BLANK & 7) * 128

        def build_idx(p, carry):
            pp = p0 + p
            t = lax.div(pp, B)
            b = lax.rem(pp, B)
            base = (b * (V * T) + lax.shift_right_logical(t, 7) * 1024
                    + lax.bitwise_and(t, 127))
            for c in range(SC16):
                cv = tgt_v[b, pl.ds(c * 16, 16)]
                off = (lax.shift_right_logical(cv, 3) * (1024 * TT)
                       + lax.shift_left(lax.bitwise_and(cv, 7), 7))
                idx_t[pl.ds(p * ROW + c * 16, 16)] = off + base
            idx_t[pl.ds(p * ROW + S, 16)] = \
                jnp.full((16,), BOFF, jnp.int32) + base
            return carry

        lax.fori_loop(0, PPW, build_idx, 0)

        pltpu.async_copy(enc_hbm.at[idx_t], val_t, sem).wait()
        pltpu.sync_copy(val_t, out_hbm.at[pl.ds(p0 * ROW, PPW * ROW)])

    out = k(enc_flat, tgt)
    return out.reshape(T, B, ROW)


def _ctc_body(lp_ref, tgt_ref, il_ref, tl_ref, out_ref, *, S):
    T, B, _ = lp_ref.shape
    tgt = tgt_ref[...]
    tgt_prev = jnp.concatenate([tgt[:, :1], tgt[:, :-1]], axis=1)
    skip = tgt != tgt_prev
    il = il_ref[...]
    tl = tl_ref[...]

    lp0 = lp_ref[0]
    neg_e = jnp.full((B, S + 1), NEG, jnp.float32)
    a_e = jnp.where(lax.broadcasted_iota(jnp.int32, (B, S + 1), 1) == 0,
                    lp0[:, S:S + 1], neg_e)
    neg_o = jnp.full((B, S), NEG, jnp.float32)
    a_o = jnp.where(lax.broadcasted_iota(jnp.int32, (B, S), 1) == 0,
                    lp0[:, :S], neg_o)

    def step(t, carry):
        a_e, a_o = carry
        lp_t = lp_ref[t]
        lpb_t = lp_t[:, S:S + 1]
        lpt_t = lp_t[:, :S]
        o_shift = jnp.concatenate(
            [jnp.full((B, 1), NEG, jnp.float32), a_o], axis=1)
        new_e = jnp.logaddexp(a_e, o_shift) + lpb_t
        s2 = jnp.where(skip, o_shift[:, :S], NEG)
        new_o = jnp.logaddexp(jnp.logaddexp(a_o, a_e[:, :S]), s2) + lpt_t
        keep = t < il
        return (jnp.where(keep, new_e, a_e), jnp.where(keep, new_o, a_o))

    a_e, a_o = lax.fori_loop(1, T, step, (a_e, a_o))

    iota_e = lax.broadcasted_iota(jnp.int32, (B, S + 1), 1)
    last_e = jnp.sum(jnp.where(iota_e == tl, a_e, 0.0), axis=1, keepdims=True)
    iota_o = lax.broadcasted_iota(jnp.int32, (B, S), 1)
    last_o = jnp.sum(jnp.where(iota_o == tl - 1, a_o, 0.0),
                     axis=1, keepdims=True)
    ctc = -jnp.logaddexp(last_e, last_o)
    out_ref[0, 0] = jnp.sum(ctc)


def _ctc_loss(lp, tgt, il, tl, S):
    T, B, _ = lp.shape
    out = pl.pallas_call(
        functools.partial(_ctc_body, S=S),
        out_specs=pl.BlockSpec(memory_space=pltpu.SMEM),
        out_shape=jax.ShapeDtypeStruct((1, 1), jnp.float32),
    )(lp, tgt, il.reshape(B, 1), tl.reshape(B, 1))
    return out[0, 0]


def kernel(inputs, labels, encoder_output, ctc_targets,
           input_lengths, target_lengths):
    B, Tdec, V = inputs.shape
    T = encoder_output.shape[0]
    S = ctc_targets.shape[1]

    tgt = ctc_targets.astype(jnp.int32)
    enc_flat = jnp.transpose(
        jnp.transpose(encoder_output, (1, 2, 0))
        .reshape(B, V // 8, 8, T // 128, 128),
        (0, 1, 3, 2, 4)).reshape(-1)
    lp = _sc_gather(enc_flat, tgt, T, B, V, S)

    scores = inputs.reshape(B * Tdec, V)
    att_total = _att_loss(scores, labels.reshape(-1))

    ctc_total = _ctc_loss(lp, tgt,
                          input_lengths.astype(jnp.int32),
                          target_lengths.astype(jnp.int32), S)

    return W * att_total + (1.0 - W) * ctc_total


# speedup vs baseline: 1.1288x; 1.1288x over previous
import functools
import numpy as np
import jax
import jax.numpy as jnp
from jax import lax
from jax.experimental import pallas as pl
from jax.experimental.pallas import tpu as pltpu
from jax.experimental.pallas import tpu_sc as plsc

BLANK = 9999
PAD = 0
LS = 0.1
CONF = 1.0 - LS
W = 0.5
NEG = -1e30

NC = 2
NS = 16
NW = NC * NS


def _att_body(scores_ref, labels_ref, out_ref, *, K, smooth):
    x = scores_ref[...]
    g = labels_ref[0]
    R, V = x.shape
    rowsum = jnp.sum(x, axis=1, keepdims=True)
    s0 = x[:, 0:1]
    iota_v = lax.broadcasted_iota(jnp.int32, (R, V), 1)
    sg = jnp.sum(jnp.where(iota_v == g, x, 0.0), axis=1, keepdims=True)
    att = jnp.where(g == PAD, 0.0,
                    K - CONF * sg - smooth * (rowsum - s0 - sg))
    part = jnp.sum(att)

    @pl.when(pl.program_id(0) == 0)
    def _():
        out_ref[0, 0] = 0.0

    out_ref[0, 0] += part


def _att_loss(scores, labels_flat):
    N, V = scores.shape
    R = 128
    nblk = N // R
    smooth = LS / (V - 2)
    K = float(CONF * np.log(CONF) + (V - 2) * smooth * np.log(smooth))
    labels3 = labels_flat.reshape(nblk, R, 1).astype(jnp.int32)
    out = pl.pallas_call(
        functools.partial(_att_body, K=K, smooth=smooth),
        grid=(nblk,),
        in_specs=[
            pl.BlockSpec((R, V), lambda i: (i, 0)),
            pl.BlockSpec((1, R, 1), lambda i: (i, 0, 0)),
        ],
        out_specs=pl.BlockSpec(memory_space=pltpu.SMEM),
        out_shape=jax.ShapeDtypeStruct((1, 1), jnp.float32),
    )(scores, labels3)
    return out[0, 0]


def _sc_gather(enc_flat, tgt, T, B, V, S):
    PPW = (T * B) // NW
    NTGT = PPW * S
    SC16 = S // 16

    mesh = plsc.VectorSubcoreMesh(core_axis_name="c", subcore_axis_name="s")

    @functools.partial(
        pl.kernel,
        out_type=[
            jax.ShapeDtypeStruct((T * B * S,), jnp.float32),
            jax.ShapeDtypeStruct((T * B,), jnp.float32),
        ],
        mesh=mesh,
        scratch_types=[
            pltpu.VMEM((B, S), jnp.int32),
            pltpu.VMEM((PPW * S,), jnp.int32),
            pltpu.VMEM((PPW * S,), jnp.float32),
            pltpu.VMEM((PPW,), jnp.int32),
            pltpu.VMEM((PPW,), jnp.float32),
            pltpu.SemaphoreType.DMA,
        ],
    )
    def k(enc_hbm, tgt_hbm, out_tgt_hbm, out_blank_hbm,
          tgt_v, idx_t, val_t, idx_b, val_b, sem):
        wid = lax.axis_index("s") * NC + lax.axis_index("c")
        p0 = wid * PPW

        pltpu.sync_copy(tgt_hbm, tgt_v)

        TT = T // 128
        BOFF = (BLANK >> 3) * (1024 * TT) + (BLANK & 7) * 128

        def build_tgt(p, carry):
            pp = p0 + p
            t = lax.div(pp, B)
            b = lax.rem(pp, B)
            base = (b * (V * T) + lax.shift_right_logical(t, 7) * 1024
                    + lax.bitwise_and(t, 127))
            for c in range(SC16):
                cv = tgt_v[b, pl.ds(c * 16, 16)]
                off = (lax.shift_right_logical(cv, 3) * (1024 * TT)
                       + lax.shift_left(lax.bitwise_and(cv, 7), 7))
                idx_t[pl.ds(p * S + c * 16, 16)] = off + base
            return carry

        lax.fori_loop(0, PPW, build_tgt, 0)

        def build_blank(c, carry):
            pvec = lax.iota(jnp.int32, 16) + (p0 + c * 16)
            bvec = lax.rem(pvec, B)
            tvec = lax.div(pvec, B)
            idx_b[pl.ds(c * 16, 16)] = (
                bvec * (V * T) + BOFF
                + lax.shift_right_logical(tvec, 7) * 1024
                + lax.bitwise_and(tvec, 127))
            return carry

        lax.fori_loop(0, PPW // 16, build_blank, 0)

        h1 = pltpu.async_copy(enc_hbm.at[idx_t], val_t, sem)
        h2 = pltpu.async_copy(enc_hbm.at[idx_b], val_b, sem)
        h1.wait()
        h2.wait()

        pltpu.sync_copy(val_t, out_tgt_hbm.at[pl.ds(p0 * S, NTGT)])
        pltpu.sync_copy(val_b, out_blank_hbm.at[pl.ds(p0, PPW)])

    out_tgt, out_blank = k(enc_flat, tgt)
    return out_tgt.reshape(T, B, S), out_blank.reshape(T, B)


def _ctc_body(lpb_ref, lpt_ref, tgt_ref, il_ref, tl_ref, out_ref):
    T, B, _ = lpb_ref.shape
    S = lpt_ref.shape[2]
    tgt = tgt_ref[...]
    tgt_prev = jnp.concatenate([tgt[:, :1], tgt[:, :-1]], axis=1)
    skip = tgt != tgt_prev
    il = il_ref[...]
    tl = tl_ref[...]

    neg_e = jnp.full((B, S + 1), NEG, jnp.float32)
    a_e = jnp.where(lax.broadcasted_iota(jnp.int32, (B, S + 1), 1) == 0,
                    lpb_ref[0], neg_e)
    neg_o = jnp.full((B, S), NEG, jnp.float32)
    a_o = jnp.where(lax.broadcasted_iota(jnp.int32, (B, S), 1) == 0,
                    lpt_ref[0], neg_o)

    def step(t, carry):
        a_e, a_o = carry
        lpb_t = lpb_ref[t]
        lpt_t = lpt_ref[t]
        o_shift = jnp.concatenate(
            [jnp.full((B, 1), NEG, jnp.float32), a_o], axis=1)
        new_e = jnp.logaddexp(a_e, o_shift) + lpb_t
        s2 = jnp.where(skip, o_shift[:, :S], NEG)
        new_o = jnp.logaddexp(jnp.logaddexp(a_o, a_e[:, :S]), s2) + lpt_t
        keep = t < il
        return (jnp.where(keep, new_e, a_e), jnp.where(keep, new_o, a_o))

    a_e, a_o = lax.fori_loop(1, T, step, (a_e, a_o))

    iota_e = lax.broadcasted_iota(jnp.int32, (B, S + 1), 1)
    last_e = jnp.sum(jnp.where(iota_e == tl, a_e, 0.0), axis=1, keepdims=True)
    iota_o = lax.broadcasted_iota(jnp.int32, (B, S), 1)
    last_o = jnp.sum(jnp.where(iota_o == tl - 1, a_o, 0.0),
                     axis=1, keepdims=True)
    ctc = -jnp.logaddexp(last_e, last_o)
    out_ref[0, 0] = jnp.sum(ctc)


def _ctc_loss(lpb, lpt, tgt, il, tl):
    T, B, S = lpt.shape
    out = pl.pallas_call(
        _ctc_body,
        out_specs=pl.BlockSpec(memory_space=pltpu.SMEM),
        out_shape=jax.ShapeDtypeStruct((1, 1), jnp.float32),
    )(lpb.reshape(T, B, 1), lpt, tgt, il.reshape(B, 1), tl.reshape(B, 1))
    return out[0, 0]


def kernel(inputs, labels, encoder_output, ctc_targets,
           input_lengths, target_lengths):
    B, Tdec, V = inputs.shape
    T = encoder_output.shape[0]
    S = ctc_targets.shape[1]

    tgt = ctc_targets.astype(jnp.int32)
    enc_flat = jnp.transpose(
        jnp.transpose(encoder_output, (1, 2, 0))
        .reshape(B, V // 8, 8, T // 128, 128),
        (0, 1, 3, 2, 4)).reshape(-1)
    lpt, lpb = _sc_gather(enc_flat, tgt, T, B, V, S)

    scores = inputs.reshape(B * Tdec, V)
    att_total = _att_loss(scores, labels.reshape(-1))

    ctc_total = _ctc_loss(lpb, lpt, tgt,
                          input_lengths.astype(jnp.int32),
                          target_lengths.astype(jnp.int32))

    return W * att_total + (1.0 - W) * ctc_total
